# Initial kernel scaffold; baseline (speedup 1.0000x reference)
import jax
import jax.numpy as jnp
from jax.experimental import pallas as pl


def kernel(nodes, samp_neighs, feat_table, og_feat_table):
    raise NotImplementedError("write your pallas kernel here")


# speedup vs baseline: 4.8330x; 4.8330x over previous
import functools

import jax
import jax.numpy as jnp
from jax import lax
from jax.experimental import pallas as pl
from jax.experimental.pallas import tpu as pltpu
from jax.experimental.pallas import tpu_sc as plsc

NC = 2
NS = 16
L = 16
NW = NC * NS


@functools.lru_cache(maxsize=None)
def _make_agg(B, S1, D, N_OG, N_FT):
    G = 4
    n_per_w = B // NW
    n_groups = n_per_w // G
    GR = G * S1
    DC = D // L

    mesh = plsc.VectorSubcoreMesh(core_axis_name="c", subcore_axis_name="s")

    @functools.partial(
        pl.kernel,
        out_type=jax.ShapeDtypeStruct((B, D), jnp.float32),
        mesh=mesh,
        scratch_types=[
            pltpu.VMEM((GR,), jnp.int32),
            pltpu.VMEM((GR, D), jnp.float32),
            pltpu.VMEM((GR, D), jnp.float32),
            pltpu.VMEM((G, D), jnp.float32),
            pltpu.VMEM((G * 32, L), jnp.float32),
            pltpu.SemaphoreType.DMA,
            pltpu.SemaphoreType.DMA,
        ],
    )
    def agg(sn_hbm, feat_hbm, og_hbm, out_hbm,
            idx_g, og_rows, feat_rows, out_v, s_vmem, sem_og, sem_ft):
        wid = lax.axis_index("s") * NC + lax.axis_index("c")
        lanes = jnp.arange(L, dtype=jnp.int32)

        def hsum(v):
            for dist in (1, 2, 4, 8):
                v = v + jnp.take_along_axis(v, lanes ^ dist, axis=0)
            return v

        def group(g, carry):
            pltpu.sync_copy(sn_hbm.at[wid, g], idx_g)
            cog = pltpu.async_copy(og_hbm.at[idx_g], og_rows, sem_og)
            cft = pltpu.async_copy(feat_hbm.at[idx_g], feat_rows, sem_ft)
            cog.wait()

            stats = []
            for n in range(G):
                rb = n * S1
                og_node = [og_rows[rb, pl.ds(c * L, L)] for c in range(DC)]

                def dotj(j, c2, _rb=rb, _og_node=og_node, _n=n):
                    rmax, ssum = c2
                    acc = _og_node[0] * og_rows[_rb + j, pl.ds(0, L)]
                    for c in range(1, DC):
                        acc = acc + _og_node[c] * og_rows[_rb + j,
                                                         pl.ds(c * L, L)]
                    sj = hsum(acc)
                    s_vmem[_n * 32 + j] = sj
                    return (jnp.maximum(rmax, sj), ssum + sj)

                rmax, ssum = lax.fori_loop(
                    0, S1, dotj,
                    (jnp.full((L,), -jnp.inf, jnp.float32),
                     jnp.zeros((L,), jnp.float32)))
                rmax = jnp.where(rmax == 0.0, jnp.float32(1.0), rmax)
                denom = jnp.float32(S1) + ssum / rmax
                stats.append((rmax, denom))

            cft.wait()

            for n in range(G):
                rb = n * S1
                rmax, denom = stats[n]

                def wsum(j, accs, _rb=rb, _n=n, _rmax=rmax, _denom=denom):
                    w = (jnp.float32(1.0) + s_vmem[_n * 32 + j] / _rmax) / _denom
                    w = jnp.where(jnp.abs(w) == jnp.inf, jnp.float32(1.0), w)
                    return tuple(
                        accs[c] + w * feat_rows[_rb + j, pl.ds(c * L, L)]
                        for c in range(DC))

                accs = lax.fori_loop(
                    0, S1, wsum,
                    tuple(jnp.zeros((L,), jnp.float32) for _ in range(DC)))
                for c in range(DC):
                    out_v[n, pl.ds(c * L, L)] = accs[c]

            base = wid * n_per_w + g * G
            pltpu.sync_copy(out_v, out_hbm.at[pl.ds(base, G)])
            return carry

        lax.fori_loop(0, n_groups, group, 0)

    return agg


def kernel(nodes, samp_neighs, feat_table, og_feat_table):
    B, S = samp_neighs.shape
    S1 = S + 1
    D = feat_table.shape[1]
    sn = jnp.concatenate(
        [nodes.reshape(-1, 1).astype(jnp.int32),
         samp_neighs.astype(jnp.int32)], axis=1)
    G = 4
    sn3 = sn.reshape(NW, (B // NW) // G, G * S1)
    agg = _make_agg(B, S1, D, og_feat_table.shape[0], feat_table.shape[0])
    return agg(sn3, feat_table, og_feat_table)


# speedup vs baseline: 6.6116x; 1.3680x over previous
import functools

import jax
import jax.numpy as jnp
from jax import lax
from jax.experimental import pallas as pl
from jax.experimental.pallas import tpu as pltpu
from jax.experimental.pallas import tpu_sc as plsc

NC = 2
NS = 16
L = 16
NW = NC * NS


@functools.lru_cache(maxsize=None)
def _make_agg(B, S1, D):
    G = 4
    n_per_w = B // NW
    n_groups = n_per_w // G
    GR = G * S1
    DC = D // L
    assert n_groups % 2 == 0 and S1 % 2 == 0

    mesh = plsc.VectorSubcoreMesh(core_axis_name="c", subcore_axis_name="s")

    @functools.partial(
        pl.kernel,
        out_type=jax.ShapeDtypeStruct((B, D), jnp.float32),
        mesh=mesh,
        scratch_types=[
            pltpu.VMEM((GR,), jnp.int32),
            pltpu.VMEM((GR,), jnp.int32),
            pltpu.VMEM((GR, D), jnp.float32),
            pltpu.VMEM((GR, D), jnp.float32),
            pltpu.VMEM((GR, D), jnp.float32),
            pltpu.VMEM((GR, D), jnp.float32),
            pltpu.VMEM((G, D), jnp.float32),
            pltpu.VMEM((G, D), jnp.float32),
            pltpu.VMEM((G * 32, L), jnp.float32),
            pltpu.SemaphoreType.DMA,
            pltpu.SemaphoreType.DMA,
            pltpu.SemaphoreType.DMA,
            pltpu.SemaphoreType.DMA,
            pltpu.SemaphoreType.DMA,
            pltpu.SemaphoreType.DMA,
        ],
    )
    def agg(sn_hbm, feat_hbm, og_hbm, out_hbm,
            idx0, idx1, og0, og1, ft0, ft1, ov0, ov1, s_vmem,
            so0, so1, sf0, sf1, su0, su1):
        idx = (idx0, idx1)
        ogr = (og0, og1)
        ftr = (ft0, ft1)
        ov = (ov0, ov1)
        sog = (so0, so1)
        sft = (sf0, sf1)
        sout = (su0, su1)

        wid = lax.axis_index("s") * NC + lax.axis_index("c")
        lanes = jnp.arange(L, dtype=jnp.int32)

        def hsum(v):
            for dist in (1, 2, 4, 8):
                v = v + jnp.take_along_axis(v, lanes ^ dist, axis=0)
            return v

        def fire(g, b):
            pltpu.sync_copy(sn_hbm.at[wid, g], idx[b])
            pltpu.async_copy(og_hbm.at[idx[b]], ogr[b], sog[b])
            pltpu.async_copy(feat_hbm.at[idx[b]], ftr[b], sft[b])

        def compute(g, b):
            base = wid * n_per_w + g * G
            pltpu.make_async_copy(og_hbm.at[idx[b]], ogr[b], sog[b]).wait()

            stats = []
            for n in range(G):
                rb = n * S1
                og_node = [ogr[b][rb, pl.ds(c * L, L)] for c in range(DC)]

                def dotj(t, c2, _rb=rb, _og=og_node, _n=n):
                    rmax, ssum = c2
                    for u in range(2):
                        j = 2 * t + u
                        row = _rb + j
                        acc = _og[0] * ogr[b][row, pl.ds(0, L)]
                        for c in range(1, DC):
                            acc = acc + _og[c] * ogr[b][row, pl.ds(c * L, L)]
                        sj = hsum(acc)
                        s_vmem[_n * 32 + j] = sj
                        rmax = jnp.maximum(rmax, sj)
                        ssum = ssum + sj
                    return (rmax, ssum)

                rmax, ssum = lax.fori_loop(
                    0, S1 // 2, dotj,
                    (jnp.full((L,), -jnp.inf, jnp.float32),
                     jnp.zeros((L,), jnp.float32)))
                rmax = jnp.where(rmax == 0.0, jnp.float32(1.0), rmax)
                denom = jnp.float32(S1) + ssum / rmax
                stats.append((rmax, denom))

            pltpu.make_async_copy(feat_hbm.at[idx[b]], ftr[b], sft[b]).wait()

            @pl.when(g >= 2)
            def _drain_out():
                pltpu.make_async_copy(
                    ov[b], out_hbm.at[pl.ds(base - 2 * G, G)], sout[b]).wait()

            for n in range(G):
                rb = n * S1
                rmax, denom = stats[n]

                def wsum(t, accs, _rb=rb, _n=n, _rmax=rmax, _denom=denom):
                    for u in range(2):
                        j = 2 * t + u
                        w = (jnp.float32(1.0)
                             + s_vmem[_n * 32 + j] / _rmax) / _denom
                        w = jnp.where(jnp.abs(w) == jnp.inf,
                                      jnp.float32(1.0), w)
                        accs = tuple(
                            accs[c] + w * ftr[b][_rb + j, pl.ds(c * L, L)]
                            for c in range(DC))
                    return accs

                accs = lax.fori_loop(
                    0, S1 // 2, wsum,
                    tuple(jnp.zeros((L,), jnp.float32) for _ in range(DC)))
                for c in range(DC):
                    ov[b][n, pl.ds(c * L, L)] = accs[c]

            pltpu.async_copy(ov[b], out_hbm.at[pl.ds(base, G)], sout[b])

        fire(0, 0)

        def body(i, carry):
            g0 = 2 * i
            fire(g0 + 1, 1)
            compute(g0, 0)

            @pl.when(i < n_groups // 2 - 1)
            def _fire_next():
                fire(g0 + 2, 0)

            compute(g0 + 1, 1)
            return carry

        lax.fori_loop(0, n_groups // 2, body, 0)

        for bb, gl in ((0, n_groups - 2), (1, n_groups - 1)):
            basel = wid * n_per_w + gl * G
            pltpu.make_async_copy(
                ov[bb], out_hbm.at[pl.ds(basel, G)], sout[bb]).wait()

    return agg


def kernel(nodes, samp_neighs, feat_table, og_feat_table):
    B, S = samp_neighs.shape
    S1 = S + 1
    D = feat_table.shape[1]
    sn = jnp.concatenate(
        [nodes.reshape(-1, 1).astype(jnp.int32),
         samp_neighs.astype(jnp.int32)], axis=1)
    G = 4
    sn3 = sn.reshape(NW, (B // NW) // G, G * S1)
    agg = _make_agg(B, S1, D)
    return agg(sn3, feat_table, og_feat_table)


# speedup vs baseline: 7.2547x; 1.0973x over previous
import functools

import jax
import jax.numpy as jnp
from jax import lax
from jax.experimental import pallas as pl
from jax.experimental.pallas import tpu as pltpu
from jax.experimental.pallas import tpu_sc as plsc

NC = 2
NS = 16
L = 16
NW = NC * NS


@functools.lru_cache(maxsize=None)
def _make_agg(B, S1, D):
    G = 4
    n_per_w = B // NW
    n_groups = n_per_w // G
    GR = G * S1
    DC = D // L
    assert n_groups % 2 == 0 and S1 % 2 == 0

    mesh = plsc.VectorSubcoreMesh(core_axis_name="c", subcore_axis_name="s")

    @functools.partial(
        pl.kernel,
        out_type=jax.ShapeDtypeStruct((B, D), jnp.float32),
        mesh=mesh,
        scratch_types=[
            pltpu.VMEM((n_groups * GR,), jnp.int32),
            pltpu.VMEM((GR, D), jnp.float32),
            pltpu.VMEM((GR, D), jnp.float32),
            pltpu.VMEM((GR, D), jnp.float32),
            pltpu.VMEM((GR, D), jnp.float32),
            pltpu.VMEM((G, D), jnp.float32),
            pltpu.VMEM((G, D), jnp.float32),
            pltpu.VMEM((G * 32 * L,), jnp.float32),
            pltpu.SemaphoreType.DMA,
            pltpu.SemaphoreType.DMA,
            pltpu.SemaphoreType.DMA,
            pltpu.SemaphoreType.DMA,
            pltpu.SemaphoreType.DMA,
            pltpu.SemaphoreType.DMA,
        ],
    )
    def agg(sn_hbm, feat_hbm, og_hbm, out_hbm,
            idx_all, og0, og1, ft0, ft1, ov0, ov1, s_vmem,
            so0, so1, sf0, sf1, su0, su1):
        ogr = (og0, og1)
        ftr = (ft0, ft1)
        ov = (ov0, ov1)
        sog = (so0, so1)
        sft = (sf0, sf1)
        sout = (su0, su1)

        wid = lax.axis_index("s") * NC + lax.axis_index("c")
        lanes = jnp.arange(L, dtype=jnp.int32)

        def hsum(v):
            for dist in (1, 2, 4, 8):
                v = v + jnp.take_along_axis(v, lanes ^ dist, axis=0)
            return v

        def fire(g, b):
            pltpu.async_copy(og_hbm.at[idx_all.at[pl.ds(g * GR, GR)]], ogr[b], sog[b])
            pltpu.async_copy(feat_hbm.at[idx_all.at[pl.ds(g * GR, GR)]], ftr[b], sft[b])

        def compute(g, b):
            base = wid * n_per_w + g * G
            pltpu.make_async_copy(
                og_hbm.at[idx_all.at[pl.ds(g * GR, GR)]], ogr[b], sog[b]).wait()

            stats = []
            for n in range(G):
                rb = n * S1
                og_node = [ogr[b][rb, pl.ds(c * L, L)] for c in range(DC)]

                def dotj(t, c2, _rb=rb, _og=og_node, _n=n):
                    rmax, ssum = c2
                    for u in range(2):
                        j = 2 * t + u
                        row = _rb + j
                        acc = _og[0] * ogr[b][row, pl.ds(0, L)]
                        for c in range(1, DC):
                            acc = acc + _og[c] * ogr[b][row, pl.ds(c * L, L)]
                        sj = hsum(acc)
                        s_vmem[pl.ds((_n * 32 + j) * L, L)] = sj
                        rmax = jnp.maximum(rmax, sj)
                        ssum = ssum + sj
                    return (rmax, ssum)

                rmax, ssum = lax.fori_loop(
                    0, S1 // 2, dotj,
                    (jnp.full((L,), -jnp.inf, jnp.float32),
                     jnp.zeros((L,), jnp.float32)))
                rmax = jnp.where(rmax == 0.0, jnp.float32(1.0), rmax)
                denom = jnp.float32(S1) + ssum / rmax
                stats.append((rmax, denom))

            pltpu.make_async_copy(
                feat_hbm.at[idx_all.at[pl.ds(g * GR, GR)]], ftr[b], sft[b]).wait()

            @pl.when(g >= 2)
            def _drain_out():
                pltpu.make_async_copy(
                    ov[b], out_hbm.at[pl.ds(base - 2 * G, G)], sout[b]).wait()

            for n in range(G):
                rb = n * S1
                rmax, denom = stats[n]

                def wsum(t, accs, _rb=rb, _n=n, _rmax=rmax, _denom=denom):
                    for u in range(2):
                        j = 2 * t + u
                        w = (jnp.float32(1.0)
                             + s_vmem[pl.ds((_n * 32 + j) * L, L)] / _rmax) / _denom
                        w = jnp.where(jnp.abs(w) == jnp.inf,
                                      jnp.float32(1.0), w)
                        accs = tuple(
                            accs[c] + w * ftr[b][_rb + j, pl.ds(c * L, L)]
                            for c in range(DC))
                    return accs

                accs = lax.fori_loop(
                    0, S1 // 2, wsum,
                    tuple(jnp.zeros((L,), jnp.float32) for _ in range(DC)))
                for c in range(DC):
                    ov[b][n, pl.ds(c * L, L)] = accs[c]

            pltpu.async_copy(ov[b], out_hbm.at[pl.ds(base, G)], sout[b])

        pltpu.sync_copy(sn_hbm.at[wid], idx_all)
        fire(0, 0)

        def body(i, carry):
            g0 = 2 * i
            fire(g0 + 1, 1)
            compute(g0, 0)

            @pl.when(i < n_groups // 2 - 1)
            def _fire_next():
                fire(g0 + 2, 0)

            compute(g0 + 1, 1)
            return carry

        lax.fori_loop(0, n_groups // 2, body, 0)

        for bb, gl in ((0, n_groups - 2), (1, n_groups - 1)):
            basel = wid * n_per_w + gl * G
            pltpu.make_async_copy(
                ov[bb], out_hbm.at[pl.ds(basel, G)], sout[bb]).wait()

    return agg


def kernel(nodes, samp_neighs, feat_table, og_feat_table):
    B, S = samp_neighs.shape
    S1 = S + 1
    D = feat_table.shape[1]
    sn = jnp.concatenate(
        [nodes.reshape(-1, 1).astype(jnp.int32),
         samp_neighs.astype(jnp.int32)], axis=1)
    G = 4
    sn3 = sn.reshape(NW, B // NW * S1)
    agg = _make_agg(B, S1, D)
    return agg(sn3, feat_table, og_feat_table)


# speedup vs baseline: 8.3890x; 1.1564x over previous
import functools

import jax
import jax.numpy as jnp
from jax import lax
from jax.experimental import pallas as pl
from jax.experimental.pallas import tpu as pltpu
from jax.experimental.pallas import tpu_sc as plsc

NC = 2
NS = 16
L = 16
NW = NC * NS


@functools.lru_cache(maxsize=None)
def _make_agg(B, S1, D):
    G = 4
    n_per_w = B // NW
    n_groups = n_per_w // G
    GR = G * S1
    DC = D // L
    assert n_groups % 2 == 0 and S1 % 2 == 0

    mesh = plsc.VectorSubcoreMesh(core_axis_name="c", subcore_axis_name="s")

    @functools.partial(
        pl.kernel,
        out_type=jax.ShapeDtypeStruct((B, D), jnp.float32),
        mesh=mesh,
        scratch_types=[
            pltpu.VMEM((n_groups * GR,), jnp.int32),
            pltpu.VMEM((GR, D), jnp.float32),
            pltpu.VMEM((GR, D), jnp.float32),
            pltpu.VMEM((GR, D), jnp.float32),
            pltpu.VMEM((GR, D), jnp.float32),
            pltpu.VMEM((G, D), jnp.float32),
            pltpu.VMEM((G, D), jnp.float32),
            pltpu.VMEM((G * 32 * L,), jnp.float32),
            pltpu.SemaphoreType.DMA,
            pltpu.SemaphoreType.DMA,
            pltpu.SemaphoreType.DMA,
            pltpu.SemaphoreType.DMA,
            pltpu.SemaphoreType.DMA,
            pltpu.SemaphoreType.DMA,
        ],
    )
    def agg(sn_hbm, feat_hbm, og_hbm, out_hbm,
            idx_all, og0, og1, ft0, ft1, ov0, ov1, s_vmem,
            so0, so1, sf0, sf1, su0, su1):
        ogr = (og0, og1)
        ftr = (ft0, ft1)
        ov = (ov0, ov1)
        sog = (so0, so1)
        sft = (sf0, sf1)
        sout = (su0, su1)

        wid = lax.axis_index("s") * NC + lax.axis_index("c")
        lanes = jnp.arange(L, dtype=jnp.int32)

        def hsum(v):
            for dist in (1, 2, 4, 8):
                v = v + jnp.take_along_axis(v, lanes ^ dist, axis=0)
            return v

        def fire(g, b):
            pltpu.async_copy(og_hbm.at[idx_all.at[pl.ds(g * GR, GR)]], ogr[b], sog[b])
            pltpu.async_copy(feat_hbm.at[idx_all.at[pl.ds(g * GR, GR)]], ftr[b], sft[b])

        def compute(g, b):
            base = wid * n_per_w + g * G
            pltpu.make_async_copy(
                og_hbm.at[idx_all.at[pl.ds(g * GR, GR)]], ogr[b], sog[b]).wait()

            stats = []
            for n in range(G):
                rb = n * S1
                og_node = [ogr[b][rb, pl.ds(c * L, L)] for c in range(DC)]

                @plsc.parallel_loop(
                    0, S1 // 2,
                    carry=(jnp.full((L,), -jnp.inf, jnp.float32),
                           jnp.zeros((L,), jnp.float32)))
                def dotj(t, c2, _rb=rb, _og=og_node, _n=n):
                    rmax, ssum = c2
                    for u in range(2):
                        j = 2 * t + u
                        row = _rb + j
                        acc0 = _og[0] * ogr[b][row, pl.ds(0, L)]
                        acc1 = _og[1] * ogr[b][row, pl.ds(L, L)]
                        for c in range(2, DC, 2):
                            acc0 = acc0 + _og[c] * ogr[b][row, pl.ds(c * L, L)]
                            acc1 = acc1 + _og[c + 1] * ogr[b][row,
                                                             pl.ds((c + 1) * L, L)]
                        sj = hsum(acc0 + acc1)
                        s_vmem[pl.ds((_n * 32 + j) * L, L)] = sj
                        rmax = jnp.maximum(rmax, sj)
                        ssum = ssum + sj
                    return (rmax, ssum)

                rmax, ssum = dotj
                rmax = jnp.where(rmax == 0.0, jnp.float32(1.0), rmax)
                denom = jnp.float32(S1) + ssum / rmax
                stats.append((rmax, denom))

            pltpu.make_async_copy(
                feat_hbm.at[idx_all.at[pl.ds(g * GR, GR)]], ftr[b], sft[b]).wait()

            @pl.when(g >= 2)
            def _drain_out():
                pltpu.make_async_copy(
                    ov[b], out_hbm.at[pl.ds(base - 2 * G, G)], sout[b]).wait()

            for n in range(G):
                rb = n * S1
                rmax, denom = stats[n]

                @plsc.parallel_loop(
                    0, S1 // 2,
                    carry=tuple(jnp.zeros((L,), jnp.float32)
                                for _ in range(DC)))
                def wsum(t, accs, _rb=rb, _n=n, _rmax=rmax, _denom=denom):
                    for u in range(2):
                        j = 2 * t + u
                        w = (jnp.float32(1.0)
                             + s_vmem[pl.ds((_n * 32 + j) * L, L)] / _rmax) / _denom
                        w = jnp.where(jnp.abs(w) == jnp.inf,
                                      jnp.float32(1.0), w)
                        accs = tuple(
                            accs[c] + w * ftr[b][_rb + j, pl.ds(c * L, L)]
                            for c in range(DC))
                    return accs

                accs = wsum
                for c in range(DC):
                    ov[b][n, pl.ds(c * L, L)] = accs[c]

            pltpu.async_copy(ov[b], out_hbm.at[pl.ds(base, G)], sout[b])

        pltpu.sync_copy(sn_hbm.at[wid], idx_all)
        fire(0, 0)

        def body(i, carry):
            g0 = 2 * i
            fire(g0 + 1, 1)
            compute(g0, 0)

            @pl.when(i < n_groups // 2 - 1)
            def _fire_next():
                fire(g0 + 2, 0)

            compute(g0 + 1, 1)
            return carry

        lax.fori_loop(0, n_groups // 2, body, 0)

        for bb, gl in ((0, n_groups - 2), (1, n_groups - 1)):
            basel = wid * n_per_w + gl * G
            pltpu.make_async_copy(
                ov[bb], out_hbm.at[pl.ds(basel, G)], sout[bb]).wait()

    return agg


def kernel(nodes, samp_neighs, feat_table, og_feat_table):
    B, S = samp_neighs.shape
    S1 = S + 1
    D = feat_table.shape[1]
    sn = jnp.concatenate(
        [nodes.reshape(-1, 1).astype(jnp.int32),
         samp_neighs.astype(jnp.int32)], axis=1)
    G = 4
    sn3 = sn.reshape(NW, B // NW * S1)
    agg = _make_agg(B, S1, D)
    return agg(sn3, feat_table, og_feat_table)
